# baseline (speedup 1.0000x reference)
import jax
import jax.numpy as jnp
from jax.experimental import pallas as pl
from jax.experimental.pallas import tpu as pltpu

EPS = 1e-08

_B, _C, _H, _W = 32, 64, 256, 256
_N = _B * _H * _W

_NH = 128
_GJ = _H // _NH


def _body(x_ref, o_ref, acc_s, acc_q):
    p = pl.program_id(0)
    i = pl.program_id(1)
    j = pl.program_id(2)

    @pl.when((p == 0) & (i == 0) & (j == 0))
    def _init():
        acc_s[...] = jnp.zeros_like(acc_s)
        acc_q[...] = jnp.zeros_like(acc_q)

    @pl.when((p == 0) & (j == 0))
    def _stats():
        xb = x_ref[...]
        acc_s[...] += jnp.sum(xb, axis=(1, 2)).reshape(_C, 1)
        acc_q[...] += jnp.sum(xb * xb, axis=(1, 2)).reshape(_C, 1)

    @pl.when(p == 1)
    def _norm():
        total = acc_s[...]
        mean = total / _N
        m2 = acc_q[...] - total * mean
        var = jnp.maximum(m2 / (_N - 1), EPS)
        rstd = jax.lax.rsqrt(var + EPS)
        xs = x_ref[:, pl.ds(j * _NH, _NH), :]
        o_ref[...] = (xs - mean[:, :, None]) * rstd[:, :, None]


def kernel(x):
    x3 = x.reshape(_B * _C, _H, _W)

    z3 = pl.pallas_call(
        _body,
        grid=(2, _B, _GJ),
        in_specs=[pl.BlockSpec((_C, _H, _W), lambda p, i, j: (i, 0, 0))],
        out_specs=pl.BlockSpec((_C, _NH, _W),
                               lambda p, i, j: (i * p, j * p, 0)),
        out_shape=jax.ShapeDtypeStruct((_B * _C, _H, _W), jnp.float32),
        scratch_shapes=[
            pltpu.VMEM((_C, 1), jnp.float32),
            pltpu.VMEM((_C, 1), jnp.float32),
        ],
        compiler_params=pltpu.CompilerParams(
            dimension_semantics=("arbitrary", "arbitrary", "arbitrary"),
            vmem_limit_bytes=56 * 1024 * 1024,
        ),
        name="welford_fused",
    )(x3)

    return z3.reshape(x.shape)


# speedup vs baseline: 1.3595x; 1.3595x over previous
import jax
import jax.numpy as jnp
from jax.experimental import pallas as pl
from jax.experimental.pallas import tpu as pltpu

EPS = 1e-08

_B, _C, _H, _W = 32, 64, 256, 256
_N = _B * _H * _W

_NH = 128
_GJ = _H // _NH
_NC = 2


def _body(x_ref, o_ref, acc_s, acc_q, cache):
    p = pl.program_id(0)
    i = pl.program_id(1)
    j = pl.program_id(2)
    s = i * _GJ + j

    @pl.when((p == 0) & (s == 0))
    def _init():
        acc_s[...] = jnp.zeros_like(acc_s)
        acc_q[...] = jnp.zeros_like(acc_q)

    @pl.when(p == 0)
    def _stats():
        xb = x_ref[...]
        acc_s[...] += jnp.sum(xb, axis=(1, 2)).reshape(_C, 1)
        acc_q[...] += jnp.sum(xb * xb, axis=(1, 2)).reshape(_C, 1)

    for k in range(_NC):
        @pl.when((p == 0) & (s == k))
        def _fill(k=k):
            cache[k] = x_ref[...]

    @pl.when(p == 1)
    def _norm():
        total = acc_s[...]
        mean = total / _N
        m2 = acc_q[...] - total * mean
        var = jnp.maximum(m2 / (_N - 1), EPS)
        rstd = jax.lax.rsqrt(var + EPS)
        mean3 = mean[:, :, None]
        rstd3 = rstd[:, :, None]

        @pl.when(s >= _NC)
        def _from_hbm():
            o_ref[...] = (x_ref[...] - mean3) * rstd3

        for k in range(_NC):
            @pl.when(s == k)
            def _from_cache(k=k):
                o_ref[...] = (cache[k] - mean3) * rstd3


def _x_index(p, i, j):
    s = i * _GJ + j
    cached = (p == 1) & (s < _NC)
    bi = jnp.where(cached, _B - 1, i)
    bj = jnp.where(cached, _GJ - 1, j)
    return (bi, bj, 0)


def kernel(x):
    x3 = x.reshape(_B * _C, _H, _W)

    z3 = pl.pallas_call(
        _body,
        grid=(2, _B, _GJ),
        in_specs=[pl.BlockSpec((_C, _NH, _W), _x_index)],
        out_specs=pl.BlockSpec((_C, _NH, _W),
                               lambda p, i, j: (i * p, j * p, 0)),
        out_shape=jax.ShapeDtypeStruct((_B * _C, _H, _W), jnp.float32),
        scratch_shapes=[
            pltpu.VMEM((_C, 1), jnp.float32),
            pltpu.VMEM((_C, 1), jnp.float32),
            pltpu.VMEM((_NC, _C, _NH, _W), jnp.float32),
        ],
        compiler_params=pltpu.CompilerParams(
            dimension_semantics=("arbitrary", "arbitrary", "arbitrary"),
            vmem_limit_bytes=58 * 1024 * 1024,
        ),
        name="welford_fused",
    )(x3)

    return z3.reshape(x.shape)
